# Initial kernel scaffold; baseline (speedup 1.0000x reference)
import jax
import jax.numpy as jnp
from jax.experimental import pallas as pl


def kernel(x, edge_index, edge_attr, batch, edge_W0, edge_b0, W1_0, b1_0, W2_0, b2_0, edge_W1, edge_b1, W1_1, b1_1, W2_1, b2_1, edge_W2, edge_b2, W1_2, b1_2, W2_2, b2_2, jk_W, jk_b, bn_gamma, bn_beta, fc2_W, fc2_b):
    raise NotImplementedError("write your pallas kernel here")


# speedup vs baseline: 1.1103x; 1.1103x over previous
import functools

import jax
import jax.numpy as jnp
from jax import lax
from jax.experimental import pallas as pl
from jax.experimental.pallas import tpu as pltpu
from jax.experimental.pallas import tpu_sc as plsc

N = 10000
E = 320000
D = 128
DE = 16
G = 128
L = 3


_BE = 4000


def _edge_linear_body(attr_ref, w_ref, out_ref):
    out_ref[0] = jnp.dot(attr_ref[...], w_ref[0],
                         preferred_element_type=jnp.float32)


def _edge_linear(edge_attr, eW):
    return pl.pallas_call(
        _edge_linear_body,
        grid=(L, E // _BE),
        in_specs=[
            pl.BlockSpec((_BE, DE), lambda l, e: (e, 0)),
            pl.BlockSpec((1, DE, D), lambda l, e: (l, 0, 0)),
        ],
        out_specs=pl.BlockSpec((1, _BE, D), lambda l, e: (l, e, 0)),
        out_shape=jax.ShapeDtypeStruct((L, E, D), jnp.float32),
    )(edge_attr, eW)


_W = 80
_ZR = 632
_NA = N + 8


_EPT = E // 32
_NCH = _EPT // _W
_LCH = 25


def _edge_pass_body(h_hbm, c_hbm, srcs_hbm, dstp0_hbm, dstp1_hbm, perm_hbm,
                    eb_hbm, zero_hbm, out_hbm,
                    src_v, dst_v, perm_v, g_v, c_v, eb_v,
                    acc_sh, sem):
    ci = lax.axis_index("c")
    s = lax.axis_index("s")
    wid = ci * 16 + s
    @pl.when(s < 15)
    def _():
        pltpu.sync_copy(zero_hbm, acc_sh.at[pl.ds(s * _ZR, _ZR)])

    @pl.when(s == 15)
    def _():
        pltpu.sync_copy(zero_hbm.at[pl.ds(0, _NA - 15 * _ZR)],
                        acc_sh.at[pl.ds(15 * _ZR, _NA - 15 * _ZR)])

    pltpu.sync_copy(eb_hbm, eb_v)
    plsc.subcore_barrier()

    def make_chunk(dst_hbm):
        def chunk(k, carry):
            base = wid * _EPT + k * _W
            pltpu.sync_copy(dst_hbm.at[pl.ds(base, _W)], dst_v)
            pltpu.sync_copy(srcs_hbm.at[pl.ds(base, _W)], src_v)
            pltpu.sync_copy(perm_hbm.at[pl.ds(base, _W)], perm_v)
            pltpu.async_copy(h_hbm.at[src_v], g_v, sem).wait()
            pltpu.async_copy(c_hbm.at[perm_v], c_v, sem).wait()

            def row(r, carry2):
                for j in range(D // 16):
                    sl = pl.ds(j * 16, 16)
                    g_v[r, sl] = jnp.maximum(
                        (g_v[r, sl] + c_v[r, sl]) + eb_v[0, sl], 0.0)
                return carry2

            lax.fori_loop(0, _W, row, 0)
            pltpu.sync_copy(g_v, acc_sh.at[dst_v], add=True)
            return carry
        return chunk

    lax.fori_loop(0, _NCH, make_chunk(dstp0_hbm), 0)
    plsc.subcore_barrier()
    lax.fori_loop(0, _LCH, make_chunk(dstp1_hbm), 0)
    plsc.subcore_barrier()

    @pl.when(s < 15)
    def _():
        pltpu.sync_copy(acc_sh.at[pl.ds(s * _ZR, _ZR)],
                        out_hbm.at[ci].at[pl.ds(s * _ZR, _ZR)])

    @pl.when(s == 15)
    def _():
        pltpu.sync_copy(acc_sh.at[pl.ds(15 * _ZR, N - 15 * _ZR)],
                        out_hbm.at[ci].at[pl.ds(15 * _ZR, N - 15 * _ZR)])


@functools.cache
def _build_edge_pass():
    return functools.partial(
        pl.kernel,
        out_type=jax.ShapeDtypeStruct((2, N, D), jnp.float32),
        mesh=plsc.VectorSubcoreMesh(core_axis_name="c", subcore_axis_name="s"),
        scratch_types=[
            pltpu.VMEM((_W,), jnp.int32),
            pltpu.VMEM((_W,), jnp.int32),
            pltpu.VMEM((_W,), jnp.int32),
            pltpu.VMEM((_W, D), jnp.float32),
            pltpu.VMEM((_W, D), jnp.float32),
            pltpu.VMEM((1, D), jnp.float32),
            pltpu.VMEM_SHARED((_NA, D), jnp.float32),
            pltpu.SemaphoreType.DMA,
        ],
    )(_edge_pass_body)


def _edge_pass(h, c_l, srcs, dst_p0, dst_p1, perm, eb, zeros):
    return _build_edge_pass()(h, c_l, srcs, dst_p0, dst_p1, perm, eb, zeros)


_BN = 2000


def _node_mlp_body(h_ref, p_ref, w1_ref, b1_ref, w2_ref, b2_ref, out_ref):
    z = h_ref[...] + p_ref[0] + p_ref[1]
    z = jnp.maximum(jnp.dot(z, w1_ref[...], preferred_element_type=jnp.float32)
                    + b1_ref[...], 0.0)
    z = jnp.dot(z, w2_ref[...], preferred_element_type=jnp.float32) + b2_ref[...]
    out_ref[...] = jnp.maximum(z, 0.0)


def _node_mlp(h, parts, W1, b1, W2, b2):
    return pl.pallas_call(
        _node_mlp_body,
        grid=(N // _BN,),
        in_specs=[
            pl.BlockSpec((_BN, D), lambda i: (i, 0)),
            pl.BlockSpec((2, _BN, D), lambda i: (0, i, 0)),
            pl.BlockSpec((D, D), lambda i: (0, 0)),
            pl.BlockSpec((1, D), lambda i: (0, 0)),
            pl.BlockSpec((D, D), lambda i: (0, 0)),
            pl.BlockSpec((1, D), lambda i: (0, 0)),
        ],
        out_specs=pl.BlockSpec((_BN, D), lambda i: (i, 0)),
        out_shape=jax.ShapeDtypeStruct((N, D), jnp.float32),
    )(h, parts, W1, b1, W2, b2)


_BH = 2000
_NH = N // _BH


def _head_body(h1_ref, h2_ref, h3_ref, j_ref, jb_ref, batch_ref,
               gam_ref, bet_ref, f2w_ref, f2b_ref, out_ref,
               sums_ref, counts_ref):
    i = pl.program_id(0)

    @pl.when(i == 0)
    def _():
        sums_ref[...] = jnp.zeros_like(sums_ref)
        counts_ref[...] = jnp.zeros_like(counts_ref)

    y = (jnp.dot(h1_ref[...], j_ref[0], preferred_element_type=jnp.float32)
         + jnp.dot(h2_ref[...], j_ref[1], preferred_element_type=jnp.float32)
         + jnp.dot(h3_ref[...], j_ref[2], preferred_element_type=jnp.float32)
         + jb_ref[...])
    ids = batch_ref[0, 0, :]
    oh = (ids[:, None] == lax.broadcasted_iota(jnp.int32, (_BH, G), 1)
          ).astype(jnp.float32)
    sums_ref[...] += lax.dot_general(
        oh, y, (((0,), (0,)), ((), ())), preferred_element_type=jnp.float32,
        precision=lax.Precision.HIGHEST)
    counts_ref[...] += lax.dot_general(
        oh, jnp.ones((_BH, D), jnp.float32), (((0,), (0,)), ((), ())),
        preferred_element_type=jnp.float32, precision=lax.Precision.HIGHEST)

    @pl.when(i == _NH - 1)
    def _():
        pooled = sums_ref[...] / jnp.maximum(counts_ref[...], 1.0)
        mean = jnp.mean(pooled, axis=0, keepdims=True)
        var = jnp.mean((pooled - mean) ** 2, axis=0, keepdims=True)
        t = (pooled - mean) / jnp.sqrt(var + 1e-5) * gam_ref[...] + bet_ref[...]
        t = jnp.where(t >= 0.0, t, 0.01 * t)
        out_ref[...] = lax.dot_general(
            f2w_ref[...], t, (((0,), (1,)), ((), ())),
            preferred_element_type=jnp.float32) + f2b_ref[...]


def _head(h1, h2, h3, jk_W, jk_b, batch3, gam, bet, f2w, f2b):
    return pl.pallas_call(
        _head_body,
        grid=(_NH,),
        in_specs=[
            pl.BlockSpec((_BH, D), lambda i: (i, 0)),
            pl.BlockSpec((_BH, D), lambda i: (i, 0)),
            pl.BlockSpec((_BH, D), lambda i: (i, 0)),
            pl.BlockSpec((L, D, D), lambda i: (0, 0, 0)),
            pl.BlockSpec((1, D), lambda i: (0, 0)),
            pl.BlockSpec((1, 1, _BH), lambda i: (i, 0, 0)),
            pl.BlockSpec((1, D), lambda i: (0, 0)),
            pl.BlockSpec((1, D), lambda i: (0, 0)),
            pl.BlockSpec((D, 1), lambda i: (0, 0)),
            pl.BlockSpec((1, 1), lambda i: (0, 0)),
        ],
        out_specs=pl.BlockSpec((1, G), lambda i: (0, 0)),
        out_shape=jax.ShapeDtypeStruct((1, G), jnp.float32),
        scratch_shapes=[
            pltpu.VMEM((G, D), jnp.float32),
            pltpu.VMEM((G, D), jnp.float32),
        ],
    )(h1, h2, h3, jk_W, jk_b, batch3, gam, bet, f2w, f2b)


def kernel(x, edge_index, edge_attr, batch,
           edge_W0, edge_b0, W1_0, b1_0, W2_0, b2_0,
           edge_W1, edge_b1, W1_1, b1_1, W2_1, b2_1,
           edge_W2, edge_b2, W1_2, b1_2, W2_2, b2_2,
           jk_W, jk_b, bn_gamma, bn_beta, fc2_W, fc2_b):
    src = edge_index[0]
    dst = edge_index[1]
    eW = jnp.stack([edge_W0, edge_W1, edge_W2])
    C = _edge_linear(edge_attr, eW)
    perm = jnp.argsort(dst).astype(jnp.int32)
    dsts = dst[perm]
    srcs = src[perm]
    prev = jnp.repeat(dsts[jnp.arange(1, 32, dtype=jnp.int32) * _EPT - 1], _EPT)
    lead = jnp.concatenate(
        [jnp.zeros((_EPT,), jnp.bool_), dsts[_EPT:] == prev])
    dst_p0 = jnp.where(lead, N, dsts)
    dst_p1 = jnp.where(lead, dsts, N)
    zeros = jnp.zeros((_ZR, D), jnp.float32)
    layer_params = [(edge_b0, W1_0, b1_0, W2_0, b2_0),
                    (edge_b1, W1_1, b1_1, W2_1, b2_1),
                    (edge_b2, W1_2, b1_2, W2_2, b2_2)]
    h = x
    hs = []
    for l in range(L):
        eb, W1, b1, W2, b2 = layer_params[l]
        parts = _edge_pass(h, C[l], srcs, dst_p0, dst_p1, perm,
                           eb.reshape(1, D), zeros)
        h = _node_mlp(h, parts, W1, b1.reshape(1, D), W2, b2.reshape(1, D))
        hs.append(h)
    out = _head(hs[0], hs[1], hs[2],
                jk_W.reshape(L, D, D), jk_b.reshape(1, D),
                batch.reshape(_NH, 1, _BH),
                bn_gamma.reshape(1, D), bn_beta.reshape(1, D),
                fc2_W, fc2_b.reshape(1, 1))
    return out.reshape(G)


# speedup vs baseline: 1.2590x; 1.1339x over previous
import functools

import jax
import jax.numpy as jnp
from jax import lax
from jax.experimental import pallas as pl
from jax.experimental.pallas import tpu as pltpu
from jax.experimental.pallas import tpu_sc as plsc

N = 10000
E = 320000
D = 128
DE = 16
G = 128
L = 3


_BE = 4000


def _edge_linear_body(attr_ref, w_ref, out_ref):
    out_ref[0] = jnp.dot(attr_ref[...], w_ref[0],
                         preferred_element_type=jnp.float32)


def _edge_linear(edge_attr, eW):
    return pl.pallas_call(
        _edge_linear_body,
        grid=(L, E // _BE),
        in_specs=[
            pl.BlockSpec((_BE, DE), lambda l, e: (e, 0)),
            pl.BlockSpec((1, DE, D), lambda l, e: (l, 0, 0)),
        ],
        out_specs=pl.BlockSpec((1, _BE, D), lambda l, e: (l, e, 0)),
        out_shape=jax.ShapeDtypeStruct((L, E, D), jnp.float32),
    )(edge_attr, eW)


_W = 80
_ZR = 632
_NA = N + 8


_EPT = E // 32
_NCH = _EPT // _W
_LCH = 13


def _edge_pass_body(h_hbm, c_hbm, srcs_hbm, dstp0_hbm, dstp1_hbm,
                    eb_hbm, zero_hbm, out_hbm,
                    src_v, dst_v, g_v, c_v, eb_v,
                    acc_sh, sem, sem2):
    ci = lax.axis_index("c")
    s = lax.axis_index("s")
    wid = ci * 16 + s
    @pl.when(s < 15)
    def _():
        pltpu.sync_copy(zero_hbm, acc_sh.at[pl.ds(s * _ZR, _ZR)])

    @pl.when(s == 15)
    def _():
        pltpu.sync_copy(zero_hbm.at[pl.ds(0, _NA - 15 * _ZR)],
                        acc_sh.at[pl.ds(15 * _ZR, _NA - 15 * _ZR)])

    pltpu.sync_copy(eb_hbm, eb_v)
    plsc.subcore_barrier()

    def make_chunk(dst_hbm):
        def chunk(k, carry):
            base = wid * _EPT + k * _W
            pltpu.sync_copy(dst_hbm.at[pl.ds(base, _W)], dst_v)
            pltpu.sync_copy(srcs_hbm.at[pl.ds(base, _W)], src_v)
            cp_c = pltpu.async_copy(c_hbm.at[pl.ds(base, _W)], c_v, sem2)
            pltpu.async_copy(h_hbm.at[src_v], g_v, sem).wait()
            cp_c.wait()

            def row(r, carry2):
                for j in range(D // 16):
                    sl = pl.ds(j * 16, 16)
                    g_v[r, sl] = jnp.maximum(
                        (g_v[r, sl] + c_v[r, sl]) + eb_v[0, sl], 0.0)
                return carry2

            lax.fori_loop(0, _W, row, 0)
            pltpu.sync_copy(g_v, acc_sh.at[dst_v], add=True)
            return carry
        return chunk

    lax.fori_loop(0, _NCH, make_chunk(dstp0_hbm), 0)
    plsc.subcore_barrier()
    lax.fori_loop(0, _LCH, make_chunk(dstp1_hbm), 0)
    plsc.subcore_barrier()

    @pl.when(s < 15)
    def _():
        pltpu.sync_copy(acc_sh.at[pl.ds(s * _ZR, _ZR)],
                        out_hbm.at[ci].at[pl.ds(s * _ZR, _ZR)])

    @pl.when(s == 15)
    def _():
        pltpu.sync_copy(acc_sh.at[pl.ds(15 * _ZR, N - 15 * _ZR)],
                        out_hbm.at[ci].at[pl.ds(15 * _ZR, N - 15 * _ZR)])


@functools.cache
def _build_edge_pass():
    return functools.partial(
        pl.kernel,
        out_type=jax.ShapeDtypeStruct((2, N, D), jnp.float32),
        mesh=plsc.VectorSubcoreMesh(core_axis_name="c", subcore_axis_name="s"),
        scratch_types=[
            pltpu.VMEM((_W,), jnp.int32),
            pltpu.VMEM((_W,), jnp.int32),
            pltpu.VMEM((_W, D), jnp.float32),
            pltpu.VMEM((_W, D), jnp.float32),
            pltpu.VMEM((1, D), jnp.float32),
            pltpu.VMEM_SHARED((_NA, D), jnp.float32),
            pltpu.SemaphoreType.DMA,
            pltpu.SemaphoreType.DMA,
        ],
    )(_edge_pass_body)


def _edge_pass(h, c_l, srcs, dst_p0, dst_p1, eb, zeros):
    return _build_edge_pass()(h, c_l, srcs, dst_p0, dst_p1, eb, zeros)


_BN = 2000


def _node_mlp_body(h_ref, p_ref, w1_ref, b1_ref, w2_ref, b2_ref, out_ref):
    z = h_ref[...] + p_ref[0] + p_ref[1]
    z = jnp.maximum(jnp.dot(z, w1_ref[...], preferred_element_type=jnp.float32)
                    + b1_ref[...], 0.0)
    z = jnp.dot(z, w2_ref[...], preferred_element_type=jnp.float32) + b2_ref[...]
    out_ref[...] = jnp.maximum(z, 0.0)


def _node_mlp(h, parts, W1, b1, W2, b2):
    return pl.pallas_call(
        _node_mlp_body,
        grid=(N // _BN,),
        in_specs=[
            pl.BlockSpec((_BN, D), lambda i: (i, 0)),
            pl.BlockSpec((2, _BN, D), lambda i: (0, i, 0)),
            pl.BlockSpec((D, D), lambda i: (0, 0)),
            pl.BlockSpec((1, D), lambda i: (0, 0)),
            pl.BlockSpec((D, D), lambda i: (0, 0)),
            pl.BlockSpec((1, D), lambda i: (0, 0)),
        ],
        out_specs=pl.BlockSpec((_BN, D), lambda i: (i, 0)),
        out_shape=jax.ShapeDtypeStruct((N, D), jnp.float32),
    )(h, parts, W1, b1, W2, b2)


_BH = 2000
_NH = N // _BH


def _head_body(h1_ref, h2_ref, h3_ref, j_ref, jb_ref, batch_ref,
               gam_ref, bet_ref, f2w_ref, f2b_ref, out_ref,
               sums_ref, counts_ref):
    i = pl.program_id(0)

    @pl.when(i == 0)
    def _():
        sums_ref[...] = jnp.zeros_like(sums_ref)
        counts_ref[...] = jnp.zeros_like(counts_ref)

    y = (jnp.dot(h1_ref[...], j_ref[0], preferred_element_type=jnp.float32)
         + jnp.dot(h2_ref[...], j_ref[1], preferred_element_type=jnp.float32)
         + jnp.dot(h3_ref[...], j_ref[2], preferred_element_type=jnp.float32)
         + jb_ref[...])
    ids = batch_ref[0, 0, :]
    oh = (ids[:, None] == lax.broadcasted_iota(jnp.int32, (_BH, G), 1)
          ).astype(jnp.float32)
    sums_ref[...] += lax.dot_general(
        oh, y, (((0,), (0,)), ((), ())), preferred_element_type=jnp.float32,
        precision=lax.Precision.HIGHEST)
    counts_ref[...] += lax.dot_general(
        oh, jnp.ones((_BH, D), jnp.float32), (((0,), (0,)), ((), ())),
        preferred_element_type=jnp.float32, precision=lax.Precision.HIGHEST)

    @pl.when(i == _NH - 1)
    def _():
        pooled = sums_ref[...] / jnp.maximum(counts_ref[...], 1.0)
        mean = jnp.mean(pooled, axis=0, keepdims=True)
        var = jnp.mean((pooled - mean) ** 2, axis=0, keepdims=True)
        t = (pooled - mean) / jnp.sqrt(var + 1e-5) * gam_ref[...] + bet_ref[...]
        t = jnp.where(t >= 0.0, t, 0.01 * t)
        out_ref[...] = lax.dot_general(
            f2w_ref[...], t, (((0,), (1,)), ((), ())),
            preferred_element_type=jnp.float32) + f2b_ref[...]


def _head(h1, h2, h3, jk_W, jk_b, batch3, gam, bet, f2w, f2b):
    return pl.pallas_call(
        _head_body,
        grid=(_NH,),
        in_specs=[
            pl.BlockSpec((_BH, D), lambda i: (i, 0)),
            pl.BlockSpec((_BH, D), lambda i: (i, 0)),
            pl.BlockSpec((_BH, D), lambda i: (i, 0)),
            pl.BlockSpec((L, D, D), lambda i: (0, 0, 0)),
            pl.BlockSpec((1, D), lambda i: (0, 0)),
            pl.BlockSpec((1, 1, _BH), lambda i: (i, 0, 0)),
            pl.BlockSpec((1, D), lambda i: (0, 0)),
            pl.BlockSpec((1, D), lambda i: (0, 0)),
            pl.BlockSpec((D, 1), lambda i: (0, 0)),
            pl.BlockSpec((1, 1), lambda i: (0, 0)),
        ],
        out_specs=pl.BlockSpec((1, G), lambda i: (0, 0)),
        out_shape=jax.ShapeDtypeStruct((1, G), jnp.float32),
        scratch_shapes=[
            pltpu.VMEM((G, D), jnp.float32),
            pltpu.VMEM((G, D), jnp.float32),
        ],
    )(h1, h2, h3, jk_W, jk_b, batch3, gam, bet, f2w, f2b)


def kernel(x, edge_index, edge_attr, batch,
           edge_W0, edge_b0, W1_0, b1_0, W2_0, b2_0,
           edge_W1, edge_b1, W1_1, b1_1, W2_1, b2_1,
           edge_W2, edge_b2, W1_2, b1_2, W2_2, b2_2,
           jk_W, jk_b, bn_gamma, bn_beta, fc2_W, fc2_b):
    src = edge_index[0]
    dst = edge_index[1]
    eW = jnp.stack([edge_W0, edge_W1, edge_W2])
    perm = jnp.argsort(dst).astype(jnp.int32)
    dsts = dst[perm]
    srcs = src[perm]
    prev = jnp.repeat(dsts[jnp.arange(1, 32, dtype=jnp.int32) * _EPT - 1], _EPT)
    lead = jnp.concatenate(
        [jnp.zeros((_EPT,), jnp.bool_), dsts[_EPT:] == prev])
    dst_p0 = jnp.where(lead, N, dsts)
    dst_p1 = jnp.where(lead, dsts, N)
    C = _edge_linear(edge_attr[perm], eW)
    zeros = jnp.zeros((_ZR, D), jnp.float32)
    layer_params = [(edge_b0, W1_0, b1_0, W2_0, b2_0),
                    (edge_b1, W1_1, b1_1, W2_1, b2_1),
                    (edge_b2, W1_2, b1_2, W2_2, b2_2)]
    h = x
    hs = []
    for l in range(L):
        eb, W1, b1, W2, b2 = layer_params[l]
        parts = _edge_pass(h, C[l], srcs, dst_p0, dst_p1,
                           eb.reshape(1, D), zeros)
        h = _node_mlp(h, parts, W1, b1.reshape(1, D), W2, b2.reshape(1, D))
        hs.append(h)
    out = _head(hs[0], hs[1], hs[2],
                jk_W.reshape(L, D, D), jk_b.reshape(1, D),
                batch.reshape(_NH, 1, _BH),
                bn_gamma.reshape(1, D), bn_beta.reshape(1, D),
                fc2_W, fc2_b.reshape(1, 1))
    return out.reshape(G)
